# Initial kernel scaffold; baseline (speedup 1.0000x reference)
import jax
import jax.numpy as jnp
from jax.experimental import pallas as pl


def kernel(xyz, par_xyz, par_feat, gen_xyz, gen_feat, params):
    raise NotImplementedError("write your pallas kernel here")


# speedup vs baseline: 1.0000x; 1.0000x over previous
import jax
import jax.numpy as jnp
from jax import lax
from jax.experimental import pallas as pl

FEAT = 256; UP = 2; KI = 8; NKNN = 16; DIM = 128; POSH = 64; ATTH = 512
B, N, NP, NG = 2, 1024, 2048, 2048


def _conv1(x, W, b):
    return jnp.einsum('oc,bcn->bon', W, x) + b[None, :, None]


def _conv2(x, W, b):
    return jnp.einsum('oc,bcnk->bonk', W, x) + b[None, :, None, None]


def _bn2(x, g, b):
    return g[None, :, None, None] * x / jnp.sqrt(1.0 + 1e-5) + b[None, :, None, None]


def _sqdist(a, b):
    return jnp.sum(a * a, -1)[:, :, None] + jnp.sum(b * b, -1)[:, None, :] - 2.0 * jnp.einsum('bnd,bmd->bnm', a, b)


def _group(feat, idx):
    return jnp.take_along_axis(feat[:, :, None, :], idx[:, None, :, :], axis=3)


def _knn_interp(qpts, kpts, vfeat, k):
    d = _sqdist(qpts, kpts)
    negv, idx = lax.top_k(-d, k)
    dist = jnp.maximum(-negv, 0.0)
    w = 1.0 / (dist + 1e-8)
    w = w / jnp.sum(w, -1, keepdims=True)
    g = _group(vfeat, idx)
    return jnp.sum(g * w[:, None, :, :], axis=-1)


def _mlp_res(x, w1, b1, w2, b2, ws, bs):
    return _conv1(jax.nn.relu(_conv1(x, w1, b1)), w2, b2) + _conv1(x, ws, bs)


def _lca(P, qpos, qfeat, kpos, kfeat):
    identity = qfeat
    q = _conv1(qfeat, P['lq_w'], P['lq_b'])
    kk = _conv1(kfeat, P['lk_w'], P['lk_b'])
    v = _conv1(kfeat, P['lv_w'], P['lv_b'])
    d = _sqdist(jnp.transpose(qpos, (0, 2, 1)), jnp.transpose(kpos, (0, 2, 1)))
    _, idx = lax.top_k(-d, NKNN)
    kg = _group(kk, idx)
    pg = _group(kpos, idx)
    pos_rel = qpos[:, :, :, None] - pg
    pe = _conv2(pos_rel, P['pm_w1'], P['pm_b1'])
    pe = jax.nn.relu(_bn2(pe, P['pm_g'], P['pm_be']))
    pe = _conv2(pe, P['pm_w2'], P['pm_b2'])
    qk = q[:, :, :, None] - kg
    a = _conv2(qk + pe, P['am_w1'], P['am_b1'])
    a = jax.nn.relu(_bn2(a, P['am_g'], P['am_be']))
    a = _conv2(a, P['am_w2'], P['am_b2'])
    a = jax.nn.softmax(a, axis=-1)
    vg = _group(v, idx) + pe
    agg = jnp.sum(a * vg, axis=-1)
    return _conv1(agg, P['le_w'], P['le_b']) + identity


def _point_shuffle(x, r):
    b, c, n = x.shape
    return jnp.transpose(x.reshape(b, c // r, r, n), (0, 1, 3, 2)).reshape(b, c // r, n * r)


def kernel(xyz, par_xyz, par_feat, gen_xyz, gen_feat, params):
    P = params
    qpts = jnp.transpose(xyz, (0, 2, 1))
    ppts = jnp.transpose(par_xyz, (0, 2, 1))
    par_interp = _knn_interp(qpts, ppts, par_feat, KI)
    gen_interp = _knn_interp(jnp.transpose(par_interp, (0, 2, 1)), jnp.transpose(par_feat, (0, 2, 1)), gen_feat, KI)
    q_raw = jnp.concatenate([xyz, par_interp, gen_interp], axis=1)
    h = _mlp_res(q_raw, P['qp1_w1'], P['qp1_b1'], P['qp1_w2'], P['qp1_b2'], P['qp1_ws'], P['qp1_bs'])
    f1 = _mlp_res(h, P['qp2_w1'], P['qp2_b1'], P['qp2_w2'], P['qp2_b2'], P['qp2_ws'], P['qp2_bs'])
    H = _lca(P, xyz, f1, xyz, f1)
    f2 = jnp.concatenate([f1, H], axis=1)
    fd = _conv1(jax.nn.relu(_conv1(f2, P['fd_w1'], P['fd_b1'])), P['fd_w2'], P['fd_b2'])
    dp = _conv1(jax.nn.relu(_conv1(jax.nn.relu(fd), P['dc_w1'], P['dc_b1'])), P['dc_w2'], P['dc_b2'])
    delta = _point_shuffle(dp, UP)
    xyz_up = jnp.repeat(xyz, UP, axis=-1) + delta
    return (xyz_up, f2, q_raw)


# speedup vs baseline: 8.0854x; 8.0852x over previous
import functools

import jax
import jax.numpy as jnp
from jax import lax
from jax.experimental import pallas as pl

FEAT = 256; UP = 2; KI = 8; NKNN = 16; DIM = 128; POSH = 64; ATTH = 512
B, N, NP, NG = 2, 1024, 2048, 2048

_INTERPRET = False


def _bf16_mm(a, b):
    return lax.dot_general(
        a.astype(jnp.bfloat16), b.astype(jnp.bfloat16),
        (((1,), (0,)), ((), ())), preferred_element_type=jnp.float32)


def _bf16_mm_nt(a, b):
    return lax.dot_general(
        a.astype(jnp.bfloat16), b.astype(jnp.bfloat16),
        (((1,), (1,)), ((), ())), preferred_element_type=jnp.float32)


def _f32_mm(a, b):
    return lax.dot_general(a, b, (((1,), (0,)), ((), ())),
                           preferred_element_type=jnp.float32,
                           precision=lax.Precision.HIGHEST)


def _interp_body(q_ref, k_ref, v_ref, o_ref):
    q = q_ref[0]
    k = k_ref[0]
    qq = jnp.sum(q * q, axis=1, keepdims=True)
    kk = jnp.sum(k * k, axis=1, keepdims=True)
    cross = _bf16_mm_nt(q, k)
    d = qq + kk.T - 2.0 * cross
    dwork = d
    t = None
    for _ in range(KI):
        t = jnp.min(dwork, axis=1, keepdims=True)
        dwork = jnp.where(dwork <= t, jnp.inf, dwork)
    w = jnp.where(d <= t, 1.0 / (jnp.maximum(d, 0.0) + 1e-8), 0.0)
    w = w / jnp.sum(w, axis=1, keepdims=True)
    o_ref[0] = _f32_mm(w, v_ref[0])


def _interp(qpts, kpts, vfeat_t):
    _, nq, dd = qpts.shape
    _, nk, cc = vfeat_t.shape
    return pl.pallas_call(
        _interp_body,
        grid=(B,),
        in_specs=[
            pl.BlockSpec((1, nq, dd), lambda b: (b, 0, 0)),
            pl.BlockSpec((1, nk, dd), lambda b: (b, 0, 0)),
            pl.BlockSpec((1, nk, cc), lambda b: (b, 0, 0)),
        ],
        out_specs=pl.BlockSpec((1, nq, cc), lambda b: (b, 0, 0)),
        out_shape=jax.ShapeDtypeStruct((B, nq, cc), jnp.float32),
        interpret=_INTERPRET,
    )(qpts, kpts, vfeat_t)


def _mlp_kernel(x_ref,
                w11_ref, b11_ref, w12_ref, b12_ref, ws1_ref, bs1_ref,
                w21_ref, b21_ref, w22_ref, b22_ref, ws2_ref, bs2_ref,
                wqkv_ref, bqkv_ref,
                f1_ref, qkv_ref):
    x = x_ref[0]
    h1 = jnp.maximum(_bf16_mm(x, w11_ref[...]) + b11_ref[...], 0.0)
    h = _bf16_mm(h1, w12_ref[...]) + b12_ref[...] + _bf16_mm(x, ws1_ref[...]) + bs1_ref[...]
    g1 = jnp.maximum(_bf16_mm(h, w21_ref[...]) + b21_ref[...], 0.0)
    f1 = _bf16_mm(g1, w22_ref[...]) + b22_ref[...] + _bf16_mm(h, ws2_ref[...]) + bs2_ref[...]
    f1_ref[0] = f1
    qkv_ref[0] = _bf16_mm(f1, wqkv_ref[...]) + bqkv_ref[...]


def _mlp(x, ws):
    nq = x.shape[1]
    in_specs = [pl.BlockSpec((1, nq, x.shape[2]), lambda b: (b, 0, 0))]
    for w in ws:
        in_specs.append(pl.BlockSpec(w.shape, lambda b, _s=w.shape: tuple(0 for _ in _s)))
    return pl.pallas_call(
        _mlp_kernel,
        grid=(B,),
        in_specs=in_specs,
        out_specs=[
            pl.BlockSpec((1, nq, FEAT), lambda b: (b, 0, 0)),
            pl.BlockSpec((1, nq, 3 * DIM), lambda b: (b, 0, 0)),
        ],
        out_shape=[
            jax.ShapeDtypeStruct((B, nq, FEAT), jnp.float32),
            jax.ShapeDtypeStruct((B, nq, 3 * DIM), jnp.float32),
        ],
        interpret=_INTERPRET,
    )(x, *ws)


_QB = 256


def _lca_kernel(qpts_ref, qkv_ref, f1_ref,
                pw1_ref, pb1_ref, pw2_ref, pb2_ref,
                aw1_ref, ab1_ref, aw2_ref, ab2_ref,
                lew_ref, leb_ref,
                out_ref):
    i = pl.program_id(1)
    pts = qpts_ref[0]
    qp = qpts_ref[0, pl.ds(i * _QB, _QB), :]
    qq = jnp.sum(qp * qp, axis=1, keepdims=True)
    kk = jnp.sum(pts * pts, axis=1, keepdims=True)
    cross = _bf16_mm_nt(qp, pts)
    d = qq + kk.T - 2.0 * cross

    qblk = qkv_ref[0, pl.ds(i * _QB, _QB), 0:DIM]
    kvtab = qkv_ref[0, :, DIM:3 * DIM]

    logits = []
    vals = []
    dwork = d
    for _ in range(NKNN):
        m = jnp.min(dwork, axis=1, keepdims=True)
        onehot = (dwork <= m).astype(jnp.float32)
        dwork = jnp.where(dwork <= m, jnp.inf, dwork)
        gkv = _f32_mm(onehot, kvtab)
        kg = gkv[:, 0:DIM]
        vg = gkv[:, DIM:2 * DIM]
        pg = _f32_mm(onehot, pts)
        pos_rel = qp - pg
        peh = jnp.maximum(_bf16_mm(pos_rel, pw1_ref[...]) + pb1_ref[...], 0.0)
        pe = _bf16_mm(peh, pw2_ref[...]) + pb2_ref[...]
        t = qblk - kg + pe
        ah = jnp.maximum(_bf16_mm(t, aw1_ref[...]) + ab1_ref[...], 0.0)
        logits.append(_bf16_mm(ah, aw2_ref[...]) + ab2_ref[...])
        vals.append(vg + pe)

    mx = logits[0]
    for r in range(1, NKNN):
        mx = jnp.maximum(mx, logits[r])
    ssum = None
    agg = None
    for r in range(NKNN):
        e = jnp.exp(logits[r] - mx)
        ssum = e if ssum is None else ssum + e
        c = e * vals[r]
        agg = c if agg is None else agg + c
    agg = agg / ssum

    f1b = f1_ref[0, pl.ds(i * _QB, _QB), :]
    out_ref[0] = _bf16_mm(agg, lew_ref[...]) + leb_ref[...] + f1b


def _lca(qpts, qkv, f1, ws):
    in_specs = [
        pl.BlockSpec((1, N, 3), lambda b, i: (b, 0, 0)),
        pl.BlockSpec((1, N, 3 * DIM), lambda b, i: (b, 0, 0)),
        pl.BlockSpec((1, N, FEAT), lambda b, i: (b, 0, 0)),
    ]
    for w in ws:
        in_specs.append(pl.BlockSpec(w.shape, lambda b, i, _s=w.shape: tuple(0 for _ in _s)))
    return pl.pallas_call(
        _lca_kernel,
        grid=(B, N // _QB),
        in_specs=in_specs,
        out_specs=pl.BlockSpec((1, _QB, FEAT), lambda b, i: (b, i, 0)),
        out_shape=jax.ShapeDtypeStruct((B, N, FEAT), jnp.float32),
        interpret=_INTERPRET,
    )(qpts, qkv, f1, *ws)


def _dec_kernel(f2_ref, w1_ref, b1_ref, w2_ref, b2_ref,
                w3_ref, b3_ref, w4_ref, b4_ref, dp_ref):
    f2 = f2_ref[0]
    h = jnp.maximum(_bf16_mm(f2, w1_ref[...]) + b1_ref[...], 0.0)
    fd = _bf16_mm(h, w2_ref[...]) + b2_ref[...]
    fr = jnp.maximum(fd, 0.0)
    g = jnp.maximum(_bf16_mm(fr, w3_ref[...]) + b3_ref[...], 0.0)
    dp_ref[0] = _bf16_mm(g, w4_ref[...]) + b4_ref[...]


def _decoder(f2, ws):
    in_specs = [pl.BlockSpec((1, N, 2 * FEAT), lambda b: (b, 0, 0))]
    for w in ws:
        in_specs.append(pl.BlockSpec(w.shape, lambda b, _s=w.shape: tuple(0 for _ in _s)))
    return pl.pallas_call(
        _dec_kernel,
        grid=(B,),
        in_specs=in_specs,
        out_specs=pl.BlockSpec((1, N, 3 * UP), lambda b: (b, 0, 0)),
        out_shape=jax.ShapeDtypeStruct((B, N, 3 * UP), jnp.float32),
        interpret=_INTERPRET,
    )(f2, *ws)


def kernel(xyz, par_xyz, par_feat, gen_xyz, gen_feat, params):
    P = params
    s = 1.0 / jnp.sqrt(jnp.float32(1.0 + 1e-5))

    qpts = jnp.transpose(xyz, (0, 2, 1))
    ppts = jnp.transpose(par_xyz, (0, 2, 1))
    pf_t = jnp.transpose(par_feat, (0, 2, 1))
    gf_t = jnp.transpose(gen_feat, (0, 2, 1))

    par_interp = _interp(qpts, ppts, pf_t)
    gen_interp = _interp(par_interp, pf_t, gf_t)

    q_raw_rows = jnp.concatenate([qpts, par_interp, gen_interp], axis=-1)

    wqkv = jnp.concatenate([P['lq_w'], P['lk_w'], P['lv_w']], axis=0).T
    bqkv = jnp.concatenate([P['lq_b'], P['lk_b'], P['lv_b']], axis=0)
    mlp_ws = (
        P['qp1_w1'].T, P['qp1_b1'], P['qp1_w2'].T, P['qp1_b2'],
        P['qp1_ws'].T, P['qp1_bs'],
        P['qp2_w1'].T, P['qp2_b1'], P['qp2_w2'].T, P['qp2_b2'],
        P['qp2_ws'].T, P['qp2_bs'],
        wqkv, bqkv,
    )
    f1, qkv = _mlp(q_raw_rows, mlp_ws)

    pg = P['pm_g'] * s
    pw1 = (P['pm_w1'] * pg[:, None]).T
    pb1 = P['pm_b1'] * pg + P['pm_be']
    ag = P['am_g'] * s
    aw1 = (P['am_w1'] * ag[:, None]).T
    ab1 = P['am_b1'] * ag + P['am_be']
    lca_ws = (
        pw1, pb1, P['pm_w2'].T, P['pm_b2'],
        aw1, ab1, P['am_w2'].T, P['am_b2'],
        P['le_w'].T, P['le_b'],
    )
    H = _lca(qpts, qkv, f1, lca_ws)

    f2_rows = jnp.concatenate([f1, H], axis=-1)
    dec_ws = (P['fd_w1'].T, P['fd_b1'], P['fd_w2'].T, P['fd_b2'],
              P['dc_w1'].T, P['dc_b1'], P['dc_w2'].T, P['dc_b2'])
    dp_rows = _decoder(f2_rows, dec_ws)

    delta = jnp.transpose(dp_rows.reshape(B, N, 3, UP), (0, 2, 1, 3)).reshape(B, 3, N * UP)
    xyz_up = jnp.repeat(xyz, UP, axis=-1) + delta

    f2 = jnp.transpose(f2_rows, (0, 2, 1))
    q_raw = jnp.transpose(q_raw_rows, (0, 2, 1))
    return (xyz_up, f2, q_raw)


# speedup vs baseline: 13.1706x; 1.6289x over previous
import functools

import jax
import jax.numpy as jnp
from jax import lax
from jax.experimental import pallas as pl

FEAT = 256; UP = 2; KI = 8; NKNN = 16; DIM = 128; POSH = 64; ATTH = 512
B, N, NP, NG = 2, 1024, 2048, 2048

_INTERPRET = False


def _bf16_mm(a, b):
    return lax.dot_general(
        a.astype(jnp.bfloat16), b.astype(jnp.bfloat16),
        (((1,), (0,)), ((), ())), preferred_element_type=jnp.float32)


def _bf16_mm_nt(a, b):
    return lax.dot_general(
        a.astype(jnp.bfloat16), b.astype(jnp.bfloat16),
        (((1,), (1,)), ((), ())), preferred_element_type=jnp.float32)


def _f32_mm(a, b):
    return lax.dot_general(a, b, (((1,), (0,)), ((), ())),
                           preferred_element_type=jnp.float32,
                           precision=lax.Precision.HIGHEST)


def _split_mm(a, b):
    ah = a.astype(jnp.bfloat16)
    al = (a - ah.astype(jnp.float32)).astype(jnp.bfloat16)
    bh = b.astype(jnp.bfloat16)
    bl = (b - bh.astype(jnp.float32)).astype(jnp.bfloat16)
    mm = lambda x, y: lax.dot_general(x, y, (((1,), (0,)), ((), ())),
                                      preferred_element_type=jnp.float32)
    return mm(ah, bh) + (mm(ah, bl) + mm(al, bh))


def _gather_mm(onehot_bf16, tab):
    th = tab.astype(jnp.bfloat16)
    tl = (tab - th.astype(jnp.float32)).astype(jnp.bfloat16)
    mm = lambda x, y: lax.dot_general(x, y, (((1,), (0,)), ((), ())),
                                      preferred_element_type=jnp.float32)
    return mm(onehot_bf16, th) + mm(onehot_bf16, tl)


def _interp_body(q_ref, k_ref, v_ref, o_ref):
    q = q_ref[0]
    k = k_ref[0]
    qq = jnp.sum(q * q, axis=1, keepdims=True)
    kk = jnp.sum(k * k, axis=1, keepdims=True)
    cross = _bf16_mm_nt(q, k)
    d = qq + kk.T - 2.0 * cross
    dwork = d
    t = None
    for _ in range(KI):
        t = jnp.min(dwork, axis=1, keepdims=True)
        dwork = jnp.where(dwork <= t, jnp.inf, dwork)
    w = jnp.where(d <= t, 1.0 / (jnp.maximum(d, 0.0) + 1e-8), 0.0)
    w = w / jnp.sum(w, axis=1, keepdims=True)
    o_ref[0] = _split_mm(w, v_ref[0])


def _interp(qpts, kpts, vfeat_t):
    _, nq, dd = qpts.shape
    _, nk, cc = vfeat_t.shape
    return pl.pallas_call(
        _interp_body,
        grid=(B,),
        in_specs=[
            pl.BlockSpec((1, nq, dd), lambda b: (b, 0, 0)),
            pl.BlockSpec((1, nk, dd), lambda b: (b, 0, 0)),
            pl.BlockSpec((1, nk, cc), lambda b: (b, 0, 0)),
        ],
        out_specs=pl.BlockSpec((1, nq, cc), lambda b: (b, 0, 0)),
        out_shape=jax.ShapeDtypeStruct((B, nq, cc), jnp.float32),
        interpret=_INTERPRET,
    )(qpts, kpts, vfeat_t)


def _mlp_kernel(x_ref,
                w11_ref, b11_ref, w12_ref, b12_ref, ws1_ref, bs1_ref,
                w21_ref, b21_ref, w22_ref, b22_ref, ws2_ref, bs2_ref,
                wqkv_ref, bqkv_ref,
                f1_ref, qkv_ref):
    x = x_ref[0]
    h1 = jnp.maximum(_bf16_mm(x, w11_ref[...]) + b11_ref[...], 0.0)
    h = _bf16_mm(h1, w12_ref[...]) + b12_ref[...] + _bf16_mm(x, ws1_ref[...]) + bs1_ref[...]
    g1 = jnp.maximum(_bf16_mm(h, w21_ref[...]) + b21_ref[...], 0.0)
    f1 = _bf16_mm(g1, w22_ref[...]) + b22_ref[...] + _bf16_mm(h, ws2_ref[...]) + bs2_ref[...]
    f1_ref[0] = f1
    qkv_ref[0] = _bf16_mm(f1, wqkv_ref[...]) + bqkv_ref[...]


def _mlp(x, ws):
    nq = x.shape[1]
    in_specs = [pl.BlockSpec((1, nq, x.shape[2]), lambda b: (b, 0, 0))]
    for w in ws:
        in_specs.append(pl.BlockSpec(w.shape, lambda b, _s=w.shape: tuple(0 for _ in _s)))
    return pl.pallas_call(
        _mlp_kernel,
        grid=(B,),
        in_specs=in_specs,
        out_specs=[
            pl.BlockSpec((1, nq, FEAT), lambda b: (b, 0, 0)),
            pl.BlockSpec((1, nq, 3 * DIM), lambda b: (b, 0, 0)),
        ],
        out_shape=[
            jax.ShapeDtypeStruct((B, nq, FEAT), jnp.float32),
            jax.ShapeDtypeStruct((B, nq, 3 * DIM), jnp.float32),
        ],
        interpret=_INTERPRET,
    )(x, *ws)


_QB = 256


def _lca_kernel(qpts_ref, qkv_ref, f1_ref,
                pw1_ref, pb1_ref, pw2_ref, pb2_ref,
                aw1_ref, ab1_ref, aw2_ref, ab2_ref,
                lew_ref, leb_ref,
                dw1_ref, db1_ref, dw2_ref, db2_ref,
                dw3_ref, db3_ref, dw4_ref, db4_ref,
                f2_ref, dp_ref):
    i = pl.program_id(1)
    pts = qpts_ref[0]
    qp = qpts_ref[0, pl.ds(i * _QB, _QB), :]
    qq = jnp.sum(qp * qp, axis=1, keepdims=True)
    kk = jnp.sum(pts * pts, axis=1, keepdims=True)
    cross = _bf16_mm_nt(qp, pts)
    d = qq + kk.T - 2.0 * cross

    qblk = qkv_ref[0, pl.ds(i * _QB, _QB), 0:DIM]
    kvtab = qkv_ref[0, :, DIM:3 * DIM]

    onehots = []
    dwork = d
    for _ in range(NKNN):
        m = jnp.min(dwork, axis=1, keepdims=True)
        sel = dwork <= m
        onehots.append(sel.astype(jnp.bfloat16))
        dwork = jnp.where(sel, jnp.inf, dwork)
    oh = jnp.concatenate(onehots, axis=0)

    gkv = _gather_mm(oh, kvtab)
    kg = gkv[:, 0:DIM]
    vg = gkv[:, DIM:2 * DIM]
    pg = _gather_mm(oh, pts)
    qp_t = jnp.concatenate([qp] * NKNN, axis=0)
    pos_rel = qp_t - pg
    peh = jnp.maximum(_bf16_mm(pos_rel, pw1_ref[...]) + pb1_ref[...], 0.0)
    pe = _bf16_mm(peh, pw2_ref[...]) + pb2_ref[...]
    qb_t = jnp.concatenate([qblk] * NKNN, axis=0)
    t = qb_t - kg + pe
    ah = jnp.maximum(_bf16_mm(t, aw1_ref[...]) + ab1_ref[...], 0.0)
    logit = _bf16_mm(ah, aw2_ref[...]) + ab2_ref[...]
    val = vg + pe

    lg = logit.reshape(NKNN, _QB, DIM)
    vl = val.reshape(NKNN, _QB, DIM)
    mx = jnp.max(lg, axis=0)
    e = jnp.exp(lg - mx[None])
    ssum = jnp.sum(e, axis=0)
    agg = jnp.sum(e * vl, axis=0) / ssum

    f1b = f1_ref[0, pl.ds(i * _QB, _QB), :]
    H = _bf16_mm(agg, lew_ref[...]) + leb_ref[...] + f1b

    f2 = jnp.concatenate([f1b, H], axis=1)
    f2_ref[0] = f2
    h = jnp.maximum(_bf16_mm(f2, dw1_ref[...]) + db1_ref[...], 0.0)
    fd = _bf16_mm(h, dw2_ref[...]) + db2_ref[...]
    fr = jnp.maximum(fd, 0.0)
    g = jnp.maximum(_bf16_mm(fr, dw3_ref[...]) + db3_ref[...], 0.0)
    dp_ref[0] = _bf16_mm(g, dw4_ref[...]) + db4_ref[...]


def _lca(qpts, qkv, f1, ws):
    in_specs = [
        pl.BlockSpec((1, N, 3), lambda b, i: (b, 0, 0)),
        pl.BlockSpec((1, N, 3 * DIM), lambda b, i: (b, 0, 0)),
        pl.BlockSpec((1, N, FEAT), lambda b, i: (b, 0, 0)),
    ]
    for w in ws:
        in_specs.append(pl.BlockSpec(w.shape, lambda b, i, _s=w.shape: tuple(0 for _ in _s)))
    return pl.pallas_call(
        _lca_kernel,
        grid=(B, N // _QB),
        in_specs=in_specs,
        out_specs=[
            pl.BlockSpec((1, _QB, 2 * FEAT), lambda b, i: (b, i, 0)),
            pl.BlockSpec((1, _QB, 3 * UP), lambda b, i: (b, i, 0)),
        ],
        out_shape=[
            jax.ShapeDtypeStruct((B, N, 2 * FEAT), jnp.float32),
            jax.ShapeDtypeStruct((B, N, 3 * UP), jnp.float32),
        ],
        interpret=_INTERPRET,
    )(qpts, qkv, f1, *ws)


def kernel(xyz, par_xyz, par_feat, gen_xyz, gen_feat, params):
    P = params
    s = 1.0 / jnp.sqrt(jnp.float32(1.0 + 1e-5))

    qpts = jnp.transpose(xyz, (0, 2, 1))
    ppts = jnp.transpose(par_xyz, (0, 2, 1))
    pf_t = jnp.transpose(par_feat, (0, 2, 1))
    gf_t = jnp.transpose(gen_feat, (0, 2, 1))

    par_interp = _interp(qpts, ppts, pf_t)
    gen_interp = _interp(par_interp, pf_t, gf_t)

    q_raw_rows = jnp.concatenate([qpts, par_interp, gen_interp], axis=-1)

    wqkv = jnp.concatenate([P['lq_w'], P['lk_w'], P['lv_w']], axis=0).T
    bqkv = jnp.concatenate([P['lq_b'], P['lk_b'], P['lv_b']], axis=0)
    mlp_ws = (
        P['qp1_w1'].T, P['qp1_b1'], P['qp1_w2'].T, P['qp1_b2'],
        P['qp1_ws'].T, P['qp1_bs'],
        P['qp2_w1'].T, P['qp2_b1'], P['qp2_w2'].T, P['qp2_b2'],
        P['qp2_ws'].T, P['qp2_bs'],
        wqkv, bqkv,
    )
    f1, qkv = _mlp(q_raw_rows, mlp_ws)

    pg = P['pm_g'] * s
    pw1 = (P['pm_w1'] * pg[:, None]).T
    pb1 = P['pm_b1'] * pg + P['pm_be']
    ag = P['am_g'] * s
    aw1 = (P['am_w1'] * ag[:, None]).T
    ab1 = P['am_b1'] * ag + P['am_be']
    lca_ws = (
        pw1, pb1, P['pm_w2'].T, P['pm_b2'],
        aw1, ab1, P['am_w2'].T, P['am_b2'],
        P['le_w'].T, P['le_b'],
        P['fd_w1'].T, P['fd_b1'], P['fd_w2'].T, P['fd_b2'],
        P['dc_w1'].T, P['dc_b1'], P['dc_w2'].T, P['dc_b2'],
    )
    f2_rows, dp_rows = _lca(qpts, qkv, f1, lca_ws)

    delta = jnp.transpose(dp_rows.reshape(B, N, 3, UP), (0, 2, 1, 3)).reshape(B, 3, N * UP)
    xyz_up = jnp.repeat(xyz, UP, axis=-1) + delta

    f2 = jnp.transpose(f2_rows, (0, 2, 1))
    q_raw = jnp.transpose(q_raw_rows, (0, 2, 1))
    return (xyz_up, f2, q_raw)
